# baseline (speedup 1.0000x reference)
import dataclasses
import functools

import jax
import jax.numpy as jnp
from jax import lax
from jax.experimental import pallas as pl
from jax.experimental.pallas import tpu as pltpu
from jax.experimental.pallas import tpu_sc as plsc

N = 10000
E = 320000
D = 128

NC, NS, L = 2, 16, 16
NW = NC * NS

PR = 320
PR_LAST = N - (NW - 1) * PR

EW = E // NW
CG = 80
NCH = EW // CG

CH = 8000
NSCH = E // CH
GW = 128
PRD = PR + 8

MLP_BLK = 2000

_SPLAT_DNUMS = lax.GatherDimensionNumbers(
    offset_dims=(), collapsed_slice_dims=(0,), start_index_map=(0,)
)


def _splat(v, l):
    idx = jnp.full((L, 1), l, jnp.int32)
    return lax.gather(v, idx, _SPLAT_DNUMS, (1,),
                      mode=lax.GatherScatterMode.PROMISE_IN_BOUNDS)


def _sc_compiler_params():
    cp = pltpu.CompilerParams()
    if "needs_layout_passes" in pltpu.CompilerParams.__dataclass_fields__:
        cp = dataclasses.replace(cp, needs_layout_passes=False)
    return cp


@functools.cache
def _mesh():
    return plsc.VectorSubcoreMesh(
        core_axis_name="c", subcore_axis_name="s", num_cores=NC, num_subcores=NS
    )


def _precompute_body(x_ref, w1a_ref, w1b_ref, b1_ref, a_ref, b_ref):
    x = x_ref[...]
    w1b = w1b_ref[...]
    wd = w1a_ref[...] - w1b
    a_ref[...] = jnp.dot(x, wd, preferred_element_type=jnp.float32) + b1_ref[...]
    b_ref[...] = jnp.dot(x, w1b, preferred_element_type=jnp.float32)


def _precompute(x, w1a, w1b, b1):
    return pl.pallas_call(
        _precompute_body,
        out_shape=(
            jax.ShapeDtypeStruct((N, D), jnp.float32),
            jax.ShapeDtypeStruct((N, D), jnp.float32),
        ),
    )(x, w1a, w1b, b1)


@functools.cache
def _gather_kernel():
  @functools.partial(
    pl.kernel,
    out_type=(
        jax.ShapeDtypeStruct((E, D), jnp.float32),
        jax.ShapeDtypeStruct((E, D), jnp.float32),
    ),
    mesh=_mesh(),
    scratch_types=[
        pltpu.VMEM((CG,), jnp.int32),
        pltpu.VMEM((CG,), jnp.int32),
        pltpu.VMEM((CG, D), jnp.float32),
        pltpu.VMEM((CG, D), jnp.float32),
        pltpu.SemaphoreType.DMA,
        pltpu.SemaphoreType.DMA,
    ],
  )
  def gather_kernel(a_hbm, b_hbm, dst_hbm, src_hbm, ad_hbm, bs_hbm,
                    di_v, si_v, ra_v, rb_v, sema, semb):
    wid = lax.axis_index("s") * NC + lax.axis_index("c")
    base0 = wid * EW

    @pl.loop(0, NCH)
    def _chunk(k):
        base = base0 + k * CG
        pltpu.sync_copy(dst_hbm.at[pl.ds(base, CG)], di_v)
        pltpu.sync_copy(src_hbm.at[pl.ds(base, CG)], si_v)
        ca = pltpu.async_copy(a_hbm.at[di_v], ra_v, sema)
        cb = pltpu.async_copy(b_hbm.at[si_v], rb_v, semb)
        ca.wait()
        cb.wait()
        pltpu.sync_copy(ra_v, ad_hbm.at[pl.ds(base, CG)])
        pltpu.sync_copy(rb_v, bs_hbm.at[pl.ds(base, CG)])

  return gather_kernel


def _mlp_body(ad_ref, bs_ref, w2_ref, b2_ref, h_ref):
    g = jnp.maximum(ad_ref[...] + bs_ref[...], 0.0)
    h = jnp.dot(g, w2_ref[...], preferred_element_type=jnp.float32) + b2_ref[...]
    h_ref[...] = jnp.maximum(h, 0.0)


def _mlp(ad, bs, w2, b2):
    nblk = E // MLP_BLK
    return pl.pallas_call(
        _mlp_body,
        grid=(nblk,),
        in_specs=[
            pl.BlockSpec((MLP_BLK, D), lambda i: (i, 0)),
            pl.BlockSpec((MLP_BLK, D), lambda i: (i, 0)),
            pl.BlockSpec((D, D), lambda i: (0, 0)),
            pl.BlockSpec((1, D), lambda i: (0, 0)),
        ],
        out_specs=pl.BlockSpec((MLP_BLK, D), lambda i: (i, 0)),
        out_shape=jax.ShapeDtypeStruct((E, D), jnp.float32),
    )(ad, bs, w2, b2)


@functools.cache
def _segmax_kernel():
  @functools.partial(
    pl.kernel,
    out_type=jax.ShapeDtypeStruct((N, D), jnp.float32),
    mesh=_mesh(),
    scratch_types=[
        pltpu.VMEM((PRD, D), jnp.float32),
        pltpu.VMEM((CH,), jnp.int32),
        pltpu.VMEM((CH + GW,), jnp.int32),
        pltpu.VMEM((CH + GW,), jnp.int32),
        pltpu.VMEM((GW, D), jnp.float32),
        pltpu.VMEM((GW, D), jnp.float32),
        pltpu.VMEM((GW,), jnp.int32),
        pltpu.VMEM((GW,), jnp.int32),
        pltpu.SemaphoreType.DMA,
        pltpu.SemaphoreType.DMA,
    ],
    compiler_params=_sc_compiler_params(),
  )
  def segmax_kernel(h_hbm, dst_hbm, out_hbm, out_v, dstv, idb, dlb,
                    rowb0, rowb1, idw0, idw1, sem0, sem1):
    wid = lax.axis_index("s") * NC + lax.axis_index("c")
    lo = wid * PR
    iota16 = lax.iota(jnp.int32, L)
    zeros16 = jnp.zeros((L,), jnp.float32)
    pad_dloc = jnp.full((L,), PR, jnp.int32)
    zeros16i = jnp.zeros((L,), jnp.int32)
    rowbs = (rowb0, rowb1)
    idws = (idw0, idw1)
    sems = (sem0, sem1)

    @pl.loop(0, PRD)
    def _zrow(r):
        for c in range(D // L):
            out_v.at[r, pl.ds(c * L, L)][...] = zeros16

    def start_gather(g, s):
        for t in range(GW // L):
            idws[s].at[pl.ds(t * L, L)][...] = idb[pl.ds(g * GW + t * L, L)]
        return pltpu.async_copy(h_hbm.at[idws[s]], rowbs[s], sems[s])

    def process(g, s):
        rowb = rowbs[s]

        @pl.loop(0, GW // L)
        def _grp(u):
            lanes = dlb[pl.ds(g * GW + u * L, L)]
            for l in range(L):
                rows = _splat(lanes, l)
                j = u * L + l
                for c in range(D // L):
                    cur = plsc.load_gather(out_v, [rows, iota16 + c * L])
                    hrow = rowb.at[j, pl.ds(c * L, L)][...]
                    plsc.store_scatter(out_v, [rows, iota16 + c * L],
                                       jnp.maximum(cur, hrow))

    @pl.loop(0, NSCH)
    def _chunk(k):
        pltpu.sync_copy(dst_hbm.at[pl.ds(k * CH, CH)], dstv)

        def scan_body(i4, cnt):
            base_i = i4 * 4 * L
            vecs = []
            for t in range(4):
                d = dstv[pl.ds(base_i + t * L, L)]
                m = (d >= lo) & (d < lo + PR)
                eid = k * CH + base_i + t * L + iota16
                vecs.append((m, eid, d))
            c = cnt
            for t in range(4):
                m, eid, d = vecs[t]
                plsc.store_compressed(idb.at[pl.ds(c, L)], eid, mask=m)
                plsc.store_compressed(dlb.at[pl.ds(c, L)], d - lo, mask=m)
                nm = plsc.all_reduce_population_count(m)
                c = c + nm[0]
            return c

        cnt = lax.fori_loop(0, CH // (4 * L), scan_body, jnp.int32(0))

        nwin = (cnt + GW - 1) // GW
        pad_end = nwin * GW
        abase = (cnt // L) * L
        for j in range(GW // L + 1):
            offs = abase + j * L + iota16
            mpad = (offs >= cnt) & (offs < pad_end)
            plsc.store_scatter(dlb, [offs], pad_dloc, mask=mpad)
            plsc.store_scatter(idb, [offs], zeros16i, mask=mpad)

        @pl.when(nwin > 0)
        def _run():
            start_gather(0, 0)

            def pbody(p, carry):
                g0 = 2 * p
                g1 = g0 + 1
                pltpu.make_async_copy(h_hbm.at[idw0], rowb0, sem0).wait()

                @pl.when(g1 < nwin)
                def _i1():
                    start_gather(g1, 1)

                process(g0, 0)

                @pl.when(g1 < nwin)
                def _p1():
                    pltpu.make_async_copy(h_hbm.at[idw1], rowb1, sem1).wait()

                    @pl.when(g1 + 1 < nwin)
                    def _i2():
                        start_gather(g1 + 1, 0)

                    process(g1, 1)

                return carry

            lax.fori_loop(0, (nwin + 1) // 2, pbody, jnp.int32(0))

    @pl.when(wid < NW - 1)
    def _wb():
        pltpu.sync_copy(out_v.at[pl.ds(0, PR)], out_hbm.at[pl.ds(lo, PR)])

    @pl.when(wid == NW - 1)
    def _wb_last():
        pltpu.sync_copy(out_v.at[pl.ds(0, PR_LAST)], out_hbm.at[pl.ds(lo, PR_LAST)])

  return segmax_kernel


def kernel(x, edge_index, W1, b1, W2, b2):
    src = edge_index[0]
    dst = edge_index[1]
    w1a = W1[:D]
    w1b = W1[D:]
    a, b = _precompute(x, w1a, w1b, b1.reshape(1, D))
    ad, bs = _gather_kernel()(a, b, dst, src)
    h = _mlp(ad, bs, W2, b2.reshape(1, D))
    return _segmax_kernel()(h, dst)


# speedup vs baseline: 2.8189x; 2.8189x over previous
import dataclasses
import functools

import jax
import jax.numpy as jnp
from jax import lax
from jax.experimental import pallas as pl
from jax.experimental.pallas import tpu as pltpu
from jax.experimental.pallas import tpu_sc as plsc

N = 10000
E = 320000
D = 128

NC, NS, L = 2, 16, 16
NW = NC * NS

PR = 320
PR_LAST = N - (NW - 1) * PR

EW = E // NW
CG = 80
NCH = EW // CG

CH = 8000
NSCH = E // CH
GW = 128
PRD = PR + 8

MLP_BLK = 2000

_SPLAT_DNUMS = lax.GatherDimensionNumbers(
    offset_dims=(), collapsed_slice_dims=(0,), start_index_map=(0,)
)


def _splat(v, l):
    idx = jnp.full((L, 1), l, jnp.int32)
    return lax.gather(v, idx, _SPLAT_DNUMS, (1,),
                      mode=lax.GatherScatterMode.PROMISE_IN_BOUNDS)


def _sc_compiler_params():
    cp = pltpu.CompilerParams()
    if "needs_layout_passes" in pltpu.CompilerParams.__dataclass_fields__:
        cp = dataclasses.replace(cp, needs_layout_passes=False)
    return cp


@functools.cache
def _mesh():
    return plsc.VectorSubcoreMesh(
        core_axis_name="c", subcore_axis_name="s", num_cores=NC, num_subcores=NS
    )


def _precompute_body(x_ref, w1a_ref, w1b_ref, b1_ref, a_ref, b_ref):
    x = x_ref[...]
    w1b = w1b_ref[...]
    wd = w1a_ref[...] - w1b
    a_ref[...] = jnp.dot(x, wd, preferred_element_type=jnp.float32) + b1_ref[...]
    b_ref[...] = jnp.dot(x, w1b, preferred_element_type=jnp.float32)


def _precompute(x, w1a, w1b, b1):
    return pl.pallas_call(
        _precompute_body,
        out_shape=(
            jax.ShapeDtypeStruct((N, D), jnp.float32),
            jax.ShapeDtypeStruct((N, D), jnp.float32),
        ),
    )(x, w1a, w1b, b1)


@functools.cache
def _gather_kernel():
  @functools.partial(
    pl.kernel,
    out_type=(
        jax.ShapeDtypeStruct((E, D), jnp.float32),
        jax.ShapeDtypeStruct((E, D), jnp.float32),
    ),
    mesh=_mesh(),
    scratch_types=[
        pltpu.VMEM((CG,), jnp.int32),
        pltpu.VMEM((CG,), jnp.int32),
        pltpu.VMEM((CG, D), jnp.float32),
        pltpu.VMEM((CG, D), jnp.float32),
        pltpu.SemaphoreType.DMA,
        pltpu.SemaphoreType.DMA,
    ],
  )
  def gather_kernel(a_hbm, b_hbm, dst_hbm, src_hbm, ad_hbm, bs_hbm,
                    di_v, si_v, ra_v, rb_v, sema, semb):
    wid = lax.axis_index("s") * NC + lax.axis_index("c")
    base0 = wid * EW

    @pl.loop(0, NCH)
    def _chunk(k):
        base = base0 + k * CG
        pltpu.sync_copy(dst_hbm.at[pl.ds(base, CG)], di_v)
        pltpu.sync_copy(src_hbm.at[pl.ds(base, CG)], si_v)
        ca = pltpu.async_copy(a_hbm.at[di_v], ra_v, sema)
        cb = pltpu.async_copy(b_hbm.at[si_v], rb_v, semb)
        ca.wait()
        cb.wait()
        pltpu.sync_copy(ra_v, ad_hbm.at[pl.ds(base, CG)])
        pltpu.sync_copy(rb_v, bs_hbm.at[pl.ds(base, CG)])

  return gather_kernel


def _mlp_body(ad_ref, bs_ref, w2_ref, b2_ref, h_ref):
    g = jnp.maximum(ad_ref[...] + bs_ref[...], 0.0)
    h = jnp.dot(g, w2_ref[...], preferred_element_type=jnp.float32) + b2_ref[...]
    h_ref[...] = jnp.maximum(h, 0.0)


def _mlp(ad, bs, w2, b2):
    nblk = E // MLP_BLK
    return pl.pallas_call(
        _mlp_body,
        grid=(nblk,),
        in_specs=[
            pl.BlockSpec((MLP_BLK, D), lambda i: (i, 0)),
            pl.BlockSpec((MLP_BLK, D), lambda i: (i, 0)),
            pl.BlockSpec((D, D), lambda i: (0, 0)),
            pl.BlockSpec((1, D), lambda i: (0, 0)),
        ],
        out_specs=pl.BlockSpec((MLP_BLK, D), lambda i: (i, 0)),
        out_shape=jax.ShapeDtypeStruct((E, D), jnp.float32),
    )(ad, bs, w2, b2)


@functools.cache
def _segmax_kernel():
  @functools.partial(
    pl.kernel,
    out_type=jax.ShapeDtypeStruct((N, D), jnp.float32),
    mesh=_mesh(),
    scratch_types=[
        pltpu.VMEM((PRD, D), jnp.float32),
        pltpu.VMEM((CH,), jnp.int32),
        pltpu.VMEM((CH + GW,), jnp.int32),
        pltpu.VMEM((CH + GW,), jnp.int32),
        pltpu.VMEM((GW, D), jnp.float32),
        pltpu.VMEM((GW, D), jnp.float32),
        pltpu.VMEM((GW,), jnp.int32),
        pltpu.VMEM((GW,), jnp.int32),
        pltpu.SemaphoreType.DMA,
        pltpu.SemaphoreType.DMA,
    ],
    compiler_params=_sc_compiler_params(),
  )
  def segmax_kernel(h_hbm, dst_hbm, out_hbm, out_v, dstv, idb, dlb,
                    rowb0, rowb1, idw0, idw1, sem0, sem1):
    wid = lax.axis_index("s") * NC + lax.axis_index("c")
    lo = wid * PR
    iota16 = lax.iota(jnp.int32, L)
    zeros16 = jnp.zeros((L,), jnp.float32)
    pad_dloc = jnp.full((L,), PR, jnp.int32)
    zeros16i = jnp.zeros((L,), jnp.int32)
    rowbs = (rowb0, rowb1)
    idws = (idw0, idw1)
    sems = (sem0, sem1)

    @pl.loop(0, PRD)
    def _zrow(r):
        for c in range(D // L):
            out_v.at[r, pl.ds(c * L, L)][...] = zeros16

    def start_gather(g, s):
        for t in range(GW // L):
            idws[s].at[pl.ds(t * L, L)][...] = idb[pl.ds(g * GW + t * L, L)]
        return pltpu.async_copy(h_hbm.at[idws[s]], rowbs[s], sems[s])

    def process(g, s):
        rowb = rowbs[s]

        @pl.loop(0, GW // L)
        def _grp(u):
            lanes = dlb[pl.ds(g * GW + u * L, L)]
            for l in range(L):
                rows = _splat(lanes, l)
                j = u * L + l
                for c in range(D // L):
                    cur = plsc.load_gather(out_v, [rows, iota16 + c * L])
                    hrow = rowb.at[j, pl.ds(c * L, L)][...]
                    plsc.store_scatter(out_v, [rows, iota16 + c * L],
                                       jnp.maximum(cur, hrow))

    @pl.loop(0, NSCH)
    def _chunk(k):
        pltpu.sync_copy(dst_hbm.at[pl.ds(k * CH, CH)], dstv)

        def scan_body(i4, cnt):
            base_i = i4 * 4 * L
            vecs = []
            for t in range(4):
                d = dstv[pl.ds(base_i + t * L, L)]
                m = (d >= lo) & (d < lo + PR)
                eid = k * CH + base_i + t * L + iota16
                vecs.append((m, eid, d))
            c = cnt
            for t in range(4):
                m, eid, d = vecs[t]
                plsc.store_compressed(idb.at[pl.ds(c, L)], eid, mask=m)
                plsc.store_compressed(dlb.at[pl.ds(c, L)], d - lo, mask=m)
                nm = plsc.all_reduce_population_count(m)
                c = c + nm[0]
            return c

        cnt = lax.fori_loop(0, CH // (4 * L), scan_body, jnp.int32(0))

        nwin = (cnt + GW - 1) // GW
        pad_end = nwin * GW
        abase = (cnt // L) * L
        for j in range(GW // L + 1):
            offs = abase + j * L + iota16
            mpad = (offs >= cnt) & (offs < pad_end)
            plsc.store_scatter(dlb, [offs], pad_dloc, mask=mpad)
            pad_eid = wid * 512 + j * L + iota16
            plsc.store_scatter(idb, [offs], pad_eid, mask=mpad)

        @pl.when(nwin > 0)
        def _run():
            start_gather(0, 0)

            def pbody(p, carry):
                g0 = 2 * p
                g1 = g0 + 1
                pltpu.make_async_copy(h_hbm.at[idw0], rowb0, sem0).wait()

                @pl.when(g1 < nwin)
                def _i1():
                    start_gather(g1, 1)

                process(g0, 0)

                @pl.when(g1 < nwin)
                def _p1():
                    pltpu.make_async_copy(h_hbm.at[idw1], rowb1, sem1).wait()

                    @pl.when(g1 + 1 < nwin)
                    def _i2():
                        start_gather(g1 + 1, 0)

                    process(g1, 1)

                return carry

            lax.fori_loop(0, (nwin + 1) // 2, pbody, jnp.int32(0))

    @pl.when(wid < NW - 1)
    def _wb():
        pltpu.sync_copy(out_v.at[pl.ds(0, PR)], out_hbm.at[pl.ds(lo, PR)])

    @pl.when(wid == NW - 1)
    def _wb_last():
        pltpu.sync_copy(out_v.at[pl.ds(0, PR_LAST)], out_hbm.at[pl.ds(lo, PR_LAST)])

  return segmax_kernel


def kernel(x, edge_index, W1, b1, W2, b2):
    src = edge_index[0]
    dst = edge_index[1]
    w1a = W1[:D]
    w1b = W1[D:]
    a, b = _precompute(x, w1a, w1b, b1.reshape(1, D))
    ad, bs = _gather_kernel()(a, b, dst, src)
    h = _mlp(ad, bs, W2, b2.reshape(1, D))
    return _segmax_kernel()(h, dst)
